# baseline (speedup 1.0000x reference)
import jax
import jax.numpy as jnp
from jax.experimental import pallas as pl
from jax.experimental.pallas import tpu as pltpu

_LANES = 128
_PAIR = 2 * _LANES
_BLOCK_ROWS = 1024


_INV_2PI = 0.15915494309189535
_2PI_HI = 6.28125
_2PI_LO = 0.0019353071795864769
_SIN_COEF = (9.99999290e-01, -1.66664832e-01, 8.33197575e-03,
             -1.97980646e-04, 2.68840882e-06, -1.99264972e-08)


def _poly_sin(x):
    k = jnp.round(x * _INV_2PI)
    r = x - k * _2PI_HI
    r = r - k * _2PI_LO
    r2 = r * r
    p = jnp.float32(_SIN_COEF[-1])
    for c in _SIN_COEF[-2::-1]:
        p = p * r2 + jnp.float32(c)
    return p * r


def _fused_step_kernel(sc_ref, x_ref, u_ref, o_ref):
    dt = sc_ref[0]
    c1 = sc_ref[1]
    c2 = sc_ref[2]
    bth = sc_ref[3]
    bdth = sc_ref[4]

    xb = x_ref[...]
    ub = u_ref[...]

    dth_nbr = pltpu.roll(xb, _PAIR - 1, 1)
    th_nbr = pltpu.roll(xb, 1, 1)

    k_iota = jax.lax.broadcasted_iota(jnp.int32, (_LANES, _PAIR), 0)
    c_iota = jax.lax.broadcasted_iota(jnp.int32, (_LANES, _PAIR), 1)
    sel = jnp.where(k_iota == c_iota // 2, 1.0, 0.0).astype(jnp.float32)
    u_exp = jax.lax.dot_general(ub, sel, (((1,), (0,)), ((), ())),
                                preferred_element_type=jnp.float32)

    th_new = xb + dt * dth_nbr + bth
    dth_new = xb + dt * (c1 * _poly_sin(th_nbr) + c2 * u_exp) + bdth

    lane = jax.lax.broadcasted_iota(jnp.int32, xb.shape, 1)
    o_ref[...] = jnp.where(lane % 2 == 0, th_new, dth_new)


def kernel(x, u, dyn_params, dyn_bias, dt):
    n = x.shape[0]
    xf = x.astype(jnp.float32).reshape(-1)
    uf = u.astype(jnp.float32).reshape(-1)

    rows = -(-n // _LANES)
    tr = min(_BLOCK_ROWS, rows)
    grid = -(-rows // tr)
    rows_pad = grid * tr
    if rows_pad * _LANES != n:
        xf = jnp.pad(xf, (0, rows_pad * _PAIR - 2 * n))
        uf = jnp.pad(uf, (0, rows_pad * _LANES - n))
    x2 = xf.reshape(rows_pad, _PAIR)
    u2 = uf.reshape(rows_pad, _LANES)

    g = jnp.float32(1.0)
    m = jnp.asarray(dyn_params[1], jnp.float32)
    l = jnp.asarray(dyn_params[2], jnp.float32)
    scalars = jnp.stack([
        jnp.asarray(dt, jnp.float32),
        3.0 * g / (2.0 * l),
        3.0 / (m * l * l),
        jnp.asarray(dyn_bias[0], jnp.float32),
        jnp.asarray(dyn_bias[1], jnp.float32),
    ])

    out2 = pl.pallas_call(
        _fused_step_kernel,
        out_shape=jax.ShapeDtypeStruct((rows_pad, _PAIR), jnp.float32),
        grid=(grid,),
        in_specs=[
            pl.BlockSpec(memory_space=pltpu.MemorySpace.SMEM),
            pl.BlockSpec((tr, _PAIR), lambda i: (i, 0)),
            pl.BlockSpec((tr, _LANES), lambda i: (i, 0)),
        ],
        out_specs=pl.BlockSpec((tr, _PAIR), lambda i: (i, 0)),
        compiler_params=pltpu.CompilerParams(
            dimension_semantics=("parallel",),
        ),
    )(scalars, x2, u2)

    return out2.reshape(-1)[: 2 * n].reshape(n, 2)


# speedup vs baseline: 153.4623x; 153.4623x over previous
import jax
import jax.numpy as jnp
from jax.experimental import pallas as pl
from jax.experimental.pallas import tpu as pltpu

_LANES = 128
_BLOCK_PAIR_ROWS = 1024

_INV_2PI = 0.15915494309189535
_2PI_HI = 6.28125
_2PI_LO = 0.0019353071795864769
_SIN_COEF = (9.99999290e-01, -1.66664832e-01, 8.33197575e-03,
             -1.97980646e-04, 2.68840882e-06, -1.99264972e-08)


def _poly_sin(x):
    k = jnp.round(x * _INV_2PI)
    r = x - k * _2PI_HI
    r = r - k * _2PI_LO
    r2 = r * r
    p = jnp.float32(_SIN_COEF[-1])
    for c in _SIN_COEF[-2::-1]:
        p = p * r2 + jnp.float32(c)
    return p * r


def _step_kernel(sc_ref, x_ref, u_ref, o_ref):
    dt = sc_ref[0]
    c1 = sc_ref[1]
    c2 = sc_ref[2]
    bth = sc_ref[3]
    bdth = sc_ref[4]

    tr = u_ref.shape[0]
    th = x_ref[pl.Slice(0, tr, 2), :]
    dth = x_ref[pl.Slice(1, tr, 2), :]
    ub = u_ref[...]

    o_ref[pl.Slice(0, tr, 2), :] = th + dt * dth + bth
    o_ref[pl.Slice(1, tr, 2), :] = dth + dt * (c1 * _poly_sin(th)
                                               + c2 * ub) + bdth


def kernel(x, u, dyn_params, dyn_bias, dt):
    n = x.shape[0]
    chunks = n // _LANES
    x2 = (x.astype(jnp.float32)
          .reshape(chunks, _LANES, 2)
          .transpose(0, 2, 1)
          .reshape(2 * chunks, _LANES))
    u2 = u.astype(jnp.float32).reshape(chunks, _LANES)

    g = jnp.float32(1.0)
    m = jnp.asarray(dyn_params[1], jnp.float32)
    l = jnp.asarray(dyn_params[2], jnp.float32)
    scalars = jnp.stack([
        jnp.asarray(dt, jnp.float32),
        3.0 * g / (2.0 * l),
        3.0 / (m * l * l),
        jnp.asarray(dyn_bias[0], jnp.float32),
        jnp.asarray(dyn_bias[1], jnp.float32),
    ])

    tr = min(_BLOCK_PAIR_ROWS, chunks)
    grid = -(-chunks // tr)

    out2 = pl.pallas_call(
        _step_kernel,
        out_shape=jax.ShapeDtypeStruct((2 * chunks, _LANES), jnp.float32),
        grid=(grid,),
        in_specs=[
            pl.BlockSpec(memory_space=pltpu.MemorySpace.SMEM),
            pl.BlockSpec((2 * tr, _LANES), lambda i: (i, 0)),
            pl.BlockSpec((tr, _LANES), lambda i: (i, 0)),
        ],
        out_specs=pl.BlockSpec((2 * tr, _LANES), lambda i: (i, 0)),
        compiler_params=pltpu.CompilerParams(
            dimension_semantics=("parallel",),
        ),
    )(scalars, x2, u2)

    return (out2.reshape(chunks, 2, _LANES)
            .transpose(0, 2, 1)
            .reshape(n, 2))
